# baseline (speedup 1.0000x reference)
import functools

import jax
import jax.numpy as jnp
from jax import lax
from jax.experimental import pallas as pl
from jax.experimental.pallas import tpu as pltpu
from jax.experimental.pallas import tpu_sc as plsc

_NB = 4
_CH = 800


@functools.lru_cache(maxsize=None)
def _make_gather(V, D, B):
    info = plsc.get_sparse_core_info()
    NC, NS = info.num_cores, info.num_subcores
    NW = NC * NS
    assert B % NW == 0
    b_per_w = B // NW
    NB, CH = _NB, _CH
    assert b_per_w % CH == 0
    n_ch = b_per_w // CH
    assert n_ch % NB == 0 and n_ch >= 2 * NB
    mesh = plsc.VectorSubcoreMesh(core_axis_name="c", subcore_axis_name="s")

    @functools.partial(
        pl.kernel,
        mesh=mesh,
        out_type=jax.ShapeDtypeStruct((B, D), jnp.uint16),
        scratch_types=[
            [pltpu.VMEM((CH,), jnp.int32)] * _NB,
            [pltpu.VMEM((CH, D), jnp.uint16)] * _NB,
            [pltpu.SemaphoreType.DMA] * _NB,
            [pltpu.SemaphoreType.DMA] * _NB,
        ],
        compiler_params=pltpu.CompilerParams(use_tc_tiling_on_sc=False),
    )
    def k(idx_hbm, table_hbm, out_hbm, idx_v, rows_v, gsems, wsems):
        wid = lax.axis_index("s") * NC + lax.axis_index("c")
        base = wid * b_per_w

        for b in range(NB):
            pltpu.sync_copy(idx_hbm.at[pl.ds(base + b * CH, CH)], idx_v[b])
            pltpu.async_copy(table_hbm.at[idx_v[b]], rows_v[b], gsems[b])

        @pl.loop(0, n_ch - NB, step=NB)
        def _ring(g):
            for b in range(NB):
                off = base + g * CH + b * CH
                pltpu.make_async_copy(
                    table_hbm.at[idx_v[b]], rows_v[b], gsems[b]
                ).wait()
                pltpu.async_copy(
                    rows_v[b], out_hbm.at[pl.ds(off, CH)], wsems[b]
                )
                nxt = off + NB * CH
                pltpu.sync_copy(idx_hbm.at[pl.ds(nxt, CH)], idx_v[b])
                pltpu.make_async_copy(
                    rows_v[b], out_hbm.at[pl.ds(base, CH)], wsems[b]
                ).wait()
                pltpu.async_copy(table_hbm.at[idx_v[b]], rows_v[b], gsems[b])

        for b in range(NB):
            off = base + (n_ch - NB + b) * CH
            pltpu.make_async_copy(
                table_hbm.at[idx_v[b]], rows_v[b], gsems[b]
            ).wait()
            pltpu.async_copy(rows_v[b], out_hbm.at[pl.ds(off, CH)], wsems[b])
        for b in range(NB):
            pltpu.make_async_copy(
                rows_v[b], out_hbm.at[pl.ds(base, CH)], wsems[b]
            ).wait()

    return k


def kernel(x, table):
    Bt, S = x.shape
    V, D = table.shape
    B = Bt * S
    xf = x.reshape(B).astype(jnp.int32)
    tu = jax.lax.bitcast_convert_type(table, jnp.uint16).reshape(V, 2 * D)
    g = _make_gather(V, 2 * D, B)(xf, tu)
    gf = jax.lax.bitcast_convert_type(g.reshape(B, D, 2), jnp.float32)
    return gf.reshape(Bt, S, D)


# speedup vs baseline: 2.8913x; 2.8913x over previous
import functools

import jax
import jax.numpy as jnp
from jax import lax
from jax.experimental import pallas as pl
from jax.experimental.pallas import tpu as pltpu
from jax.experimental.pallas import tpu_sc as plsc

_NB = 4
_CH = 800


@functools.lru_cache(maxsize=None)
def _make_gather(V, D, B):
    info = plsc.get_sparse_core_info()
    NC, NS = info.num_cores, info.num_subcores
    NW = NC * NS
    assert B % NW == 0
    b_per_w = B // NW
    NB, CH = _NB, _CH
    assert b_per_w % CH == 0
    n_ch = b_per_w // CH
    assert n_ch % NB == 0 and n_ch >= 2 * NB
    mesh = plsc.VectorSubcoreMesh(core_axis_name="c", subcore_axis_name="s")

    @functools.partial(
        pl.kernel,
        mesh=mesh,
        out_type=jax.ShapeDtypeStruct((B, D), jnp.int32),
        scratch_types=[
            [pltpu.VMEM((CH,), jnp.int32)] * _NB,
            [pltpu.VMEM((CH, D), jnp.int32)] * _NB,
            [pltpu.SemaphoreType.DMA] * _NB,
            [pltpu.SemaphoreType.DMA] * _NB,
        ],
        compiler_params=pltpu.CompilerParams(use_tc_tiling_on_sc=False),
    )
    def k(idx_hbm, table_hbm, out_hbm, idx_v, rows_v, gsems, wsems):
        wid = lax.axis_index("s") * NC + lax.axis_index("c")
        base = wid * b_per_w

        for b in range(NB):
            pltpu.sync_copy(idx_hbm.at[pl.ds(base + b * CH, CH)], idx_v[b])
            pltpu.async_copy(table_hbm.at[idx_v[b]], rows_v[b], gsems[b])

        @pl.loop(0, n_ch - NB, step=NB)
        def _ring(g):
            for b in range(NB):
                off = base + g * CH + b * CH
                pltpu.make_async_copy(
                    table_hbm.at[idx_v[b]], rows_v[b], gsems[b]
                ).wait()
                pltpu.async_copy(
                    rows_v[b], out_hbm.at[pl.ds(off, CH)], wsems[b]
                )
                nxt = off + NB * CH
                pltpu.sync_copy(idx_hbm.at[pl.ds(nxt, CH)], idx_v[b])
                pltpu.make_async_copy(
                    rows_v[b], out_hbm.at[pl.ds(base, CH)], wsems[b]
                ).wait()
                pltpu.async_copy(table_hbm.at[idx_v[b]], rows_v[b], gsems[b])

        for b in range(NB):
            off = base + (n_ch - NB + b) * CH
            pltpu.make_async_copy(
                table_hbm.at[idx_v[b]], rows_v[b], gsems[b]
            ).wait()
            pltpu.async_copy(rows_v[b], out_hbm.at[pl.ds(off, CH)], wsems[b])
        for b in range(NB):
            pltpu.make_async_copy(
                rows_v[b], out_hbm.at[pl.ds(base, CH)], wsems[b]
            ).wait()

    return k


def kernel(x, table):
    Bt, S = x.shape
    V, D = table.shape
    B = Bt * S
    xf = x.reshape(B).astype(jnp.int32)
    ti = jax.lax.bitcast_convert_type(table, jnp.int32)
    g = _make_gather(V, D, B)(xf, ti)
    return jax.lax.bitcast_convert_type(g, jnp.float32).reshape(Bt, S, D)
